# Initial kernel scaffold; baseline (speedup 1.0000x reference)
import functools

import jax
import jax.numpy as jnp
from jax import lax
from jax.experimental import pallas as pl
from jax.experimental.pallas import tpu as pltpu
from jax.experimental.pallas import tpu_sc as plsc

W = 2048
H = 2048
C = 4
N = 2000000

NC = 2
NS = 16
NW = NC * NS
VEC = 16

CHUNK = 2000
NVEC = CHUNK // VEC
NCHUNKS = N // CHUNK
ITERS = (NCHUNKS + NW - 1) // NW


def _search(c_ref, q, n):
  lo = jnp.zeros((VEC,), jnp.int32)
  step = n >> 1
  while step:
    m = lo + step
    v = plsc.load_gather(c_ref, [m - 1])
    lo = jnp.where(v <= q, m, lo)
    step >>= 1
  cmax = plsc.load_gather(c_ref, [jnp.full((VEC,), n - 1, jnp.int32)])
  cnt = jnp.where(cmax <= q, n, lo)
  xl = cnt - 1
  valid = (xl >= 0) & (xl <= n - 2)
  xlc = jnp.clip(xl, 0, n - 2)
  cl = plsc.load_gather(c_ref, [xlc])
  cu = plsc.load_gather(c_ref, [xlc + 1])
  rd = 1.0 / (cu - cl)
  return xlc, (cu - q) * rd, (q - cl) * rd, valid


def _make_kernel():
  mesh = plsc.VectorSubcoreMesh(core_axis_name="c", subcore_axis_name="s")

  @functools.partial(
      pl.kernel,
      out_type=jax.ShapeDtypeStruct((C, N), jnp.float32),
      mesh=mesh,
      scratch_types=[
          pltpu.VMEM((W,), jnp.float32),
          pltpu.VMEM((H,), jnp.float32),
          pltpu.VMEM((CHUNK,), jnp.float32),
          pltpu.VMEM((CHUNK,), jnp.float32),
          pltpu.VMEM((NVEC, VEC), jnp.int32),
          pltpu.VMEM((NVEC, VEC), jnp.int32),
          pltpu.VMEM((CHUNK,), jnp.float32),
          pltpu.VMEM((CHUNK,), jnp.float32),
          pltpu.VMEM((CHUNK,), jnp.float32),
          pltpu.VMEM((CHUNK,), jnp.float32),
          pltpu.VMEM((CHUNK,), jnp.float32),
          pltpu.VMEM((NVEC, VEC, 8), jnp.float32),
          pltpu.VMEM((NVEC, VEC, 8), jnp.float32),
          pltpu.VMEM((C, CHUNK), jnp.float32),
          pltpu.SemaphoreType.DMA,
      ],
  )
  def kern(cx_hbm, cy_hbm, xq_hbm, yq_hbm, rows8_hbm, out_hbm,
           cx, cy, xq, yq, idx0, idx1, wxl, wxu, wyl, wyu, msk, g0, g1, ob,
           sem):
    wid = lax.axis_index("s") * NC + lax.axis_index("c")
    pltpu.sync_copy(cx_hbm, cx)
    pltpu.sync_copy(cy_hbm, cy)

    def chunk_body(i, _):
      cid = i * NW + wid

      @pl.when(cid < NCHUNKS)
      def _():
        base = cid * CHUNK
        pltpu.sync_copy(xq_hbm.at[pl.ds(base, CHUNK)], xq)
        pltpu.sync_copy(yq_hbm.at[pl.ds(base, CHUNK)], yq)

        def vec_body(v, _):
          off = v * VEC
          qx = xq[pl.ds(off, VEC)]
          qy = yq[pl.ds(off, VEC)]
          xlc, xw_l, xw_u, mx = _search(cx, qx, W)
          ylc, yw_l, yw_u, my = _search(cy, qy, H)
          p0 = ylc * W + xlc
          idx0[v] = p0
          idx1[v] = p0 + W
          wxl[pl.ds(off, VEC)] = xw_l
          wxu[pl.ds(off, VEC)] = xw_u
          wyl[pl.ds(off, VEC)] = yw_l
          wyu[pl.ds(off, VEC)] = yw_u
          msk[pl.ds(off, VEC)] = jnp.where(mx & my, 1.0, 0.0)
          return 0

        lax.fori_loop(0, NVEC, vec_body, 0)

        cp0 = pltpu.async_copy(rows8_hbm.at[idx0], g0, sem)
        cp1 = pltpu.async_copy(rows8_hbm.at[idx1], g1, sem)
        cp0.wait()
        cp1.wait()

        lanes = lax.iota(jnp.int32, VEC)

        def red_body(v, _):
          off = v * VEC
          vfull = jnp.full((VEC,), 0, jnp.int32) + v
          axl = wxl[pl.ds(off, VEC)]
          axu = wxu[pl.ds(off, VEC)]
          ayl = wyl[pl.ds(off, VEC)]
          ayu = wyu[pl.ds(off, VEC)]
          m = msk[pl.ds(off, VEC)]
          for c in range(C):
            cf = jnp.full((VEC,), c, jnp.int32)
            cf4 = jnp.full((VEC,), c + 4, jnp.int32)
            r00 = plsc.load_gather(g0, [vfull, lanes, cf])
            r01 = plsc.load_gather(g0, [vfull, lanes, cf4])
            r10 = plsc.load_gather(g1, [vfull, lanes, cf])
            r11 = plsc.load_gather(g1, [vfull, lanes, cf4])
            o = ayl * (axl * r00 + axu * r01) + ayu * (axl * r10 + axu * r11)
            o = jnp.where(m != 0.0, o, 0.0)
            ob[c, pl.ds(off, VEC)] = o
          return 0

        lax.fori_loop(0, NVEC, red_body, 0)

        for c in range(C):
          pltpu.sync_copy(ob.at[c], out_hbm.at[c, pl.ds(base, CHUNK)])

      return 0

    lax.fori_loop(0, ITERS, chunk_body, 0)

  return kern


_interp = _make_kernel()


@jax.jit
def kernel(x_coords, y_coords, x_query, y_query, z):
  zrs = z.reshape(C, H * W).T
  rows8 = jnp.concatenate([zrs, jnp.roll(zrs, -1, axis=0)], axis=1)
  return _interp(x_coords, y_coords, x_query, y_query, rows8)


# speedup vs baseline: 108.1639x; 108.1639x over previous
import functools

import jax
import jax.numpy as jnp
from jax import lax
from jax.experimental import pallas as pl
from jax.experimental.pallas import tpu as pltpu
from jax.experimental.pallas import tpu_sc as plsc

W = 2048
H = 2048
C = 4
N = 2000000

NC = 2
NS = 16
NW = NC * NS
VEC = 16

CHUNK = 2000
NVEC = CHUNK // VEC
NCHUNKS = N // CHUNK
ITERS = (NCHUNKS + NW - 1) // NW


def _search(c_ref, q, n):
  lo = jnp.zeros((VEC,), jnp.int32)
  step = n >> 1
  while step:
    m = lo + step
    v = plsc.load_gather(c_ref, [m - 1])
    lo = jnp.where(v <= q, m, lo)
    step >>= 1
  cmax = plsc.load_gather(c_ref, [jnp.full((VEC,), n - 1, jnp.int32)])
  cnt = jnp.where(cmax <= q, n, lo)
  xl = cnt - 1
  valid = (xl >= 0) & (xl <= n - 2)
  xlc = jnp.clip(xl, 0, n - 2)
  cl = plsc.load_gather(c_ref, [xlc])
  cu = plsc.load_gather(c_ref, [xlc + 1])
  rd = 1.0 / (cu - cl)
  return xlc, (cu - q) * rd, (q - cl) * rd, valid


def _make_kernel():
  mesh = plsc.VectorSubcoreMesh(core_axis_name="c", subcore_axis_name="s")

  @functools.partial(
      pl.kernel,
      out_type=jax.ShapeDtypeStruct((C * N,), jnp.float32),
      mesh=mesh,
      compiler_params=pltpu.CompilerParams(
          needs_layout_passes=False, use_tc_tiling_on_sc=False),
      scratch_types=[
          pltpu.VMEM((W,), jnp.float32),
          pltpu.VMEM((H,), jnp.float32),
          pltpu.VMEM((CHUNK,), jnp.float32),
          pltpu.VMEM((CHUNK,), jnp.float32),
          pltpu.VMEM((CHUNK,), jnp.int32),
          pltpu.VMEM((CHUNK,), jnp.int32),
          pltpu.VMEM((CHUNK,), jnp.float32),
          pltpu.VMEM((CHUNK,), jnp.float32),
          pltpu.VMEM((CHUNK,), jnp.float32),
          pltpu.VMEM((CHUNK,), jnp.float32),
          pltpu.VMEM((CHUNK,), jnp.float32),
          pltpu.VMEM((CHUNK, 8), jnp.float32),
          pltpu.VMEM((CHUNK, 8), jnp.float32),
          pltpu.VMEM((C, CHUNK), jnp.float32),
          pltpu.SemaphoreType.DMA,
      ],
  )
  def kern(cx_hbm, cy_hbm, xq_hbm, yq_hbm, rows8_hbm, out_hbm,
           cx, cy, xq, yq, idx0, idx1, wxl, wxu, wyl, wyu, msk, g0, g1, ob,
           sem):
    wid = lax.axis_index("s") * NC + lax.axis_index("c")
    pltpu.sync_copy(cx_hbm, cx)
    pltpu.sync_copy(cy_hbm, cy)

    def chunk_body(i, _):
      cid = i * NW + wid

      @pl.when(cid < NCHUNKS)
      def _():
        base = cid * CHUNK
        pltpu.sync_copy(xq_hbm.at[pl.ds(base, CHUNK)], xq)
        pltpu.sync_copy(yq_hbm.at[pl.ds(base, CHUNK)], yq)

        def vec_body(v, _):
          off = v * VEC
          qx = xq[pl.ds(off, VEC)]
          qy = yq[pl.ds(off, VEC)]
          xlc, xw_l, xw_u, mx = _search(cx, qx, W)
          ylc, yw_l, yw_u, my = _search(cy, qy, H)
          p0 = ylc * W + xlc
          idx0[pl.ds(off, VEC)] = p0
          idx1[pl.ds(off, VEC)] = p0 + W
          wxl[pl.ds(off, VEC)] = xw_l
          wxu[pl.ds(off, VEC)] = xw_u
          wyl[pl.ds(off, VEC)] = yw_l
          wyu[pl.ds(off, VEC)] = yw_u
          msk[pl.ds(off, VEC)] = jnp.where(mx & my, 1.0, 0.0)
          return 0

        lax.fori_loop(0, NVEC, vec_body, 0)

        cp0 = pltpu.async_copy(rows8_hbm.at[idx0], g0, sem)
        cp1 = pltpu.async_copy(rows8_hbm.at[idx1], g1, sem)
        cp0.wait()
        cp1.wait()

        lanes = lax.iota(jnp.int32, VEC)

        def red_body(v, _):
          off = v * VEC
          qidx = lanes + off
          axl = wxl[pl.ds(off, VEC)]
          axu = wxu[pl.ds(off, VEC)]
          ayl = wyl[pl.ds(off, VEC)]
          ayu = wyu[pl.ds(off, VEC)]
          m = msk[pl.ds(off, VEC)]
          for c in range(C):
            cf = jnp.full((VEC,), c, jnp.int32)
            cf4 = jnp.full((VEC,), c + 4, jnp.int32)
            r00 = plsc.load_gather(g0, [qidx, cf])
            r01 = plsc.load_gather(g0, [qidx, cf4])
            r10 = plsc.load_gather(g1, [qidx, cf])
            r11 = plsc.load_gather(g1, [qidx, cf4])
            o = ayl * (axl * r00 + axu * r01) + ayu * (axl * r10 + axu * r11)
            o = jnp.where(m != 0.0, o, 0.0)
            ob[c, pl.ds(off, VEC)] = o
          return 0

        lax.fori_loop(0, NVEC, red_body, 0)

        for c in range(C):
          pltpu.sync_copy(ob.at[c], out_hbm.at[pl.ds(c * N + base, CHUNK)])

      return 0

    lax.fori_loop(0, ITERS, chunk_body, 0)

  return kern


_interp = _make_kernel()


@jax.jit
def kernel(x_coords, y_coords, x_query, y_query, z):
  zrs = z.reshape(C, H * W).T
  rows8 = jnp.concatenate([zrs, jnp.roll(zrs, -1, axis=0)], axis=1)
  out = _interp(x_coords, y_coords, x_query, y_query, rows8)
  return out.reshape(C, N)


# speedup vs baseline: 129.2080x; 1.1946x over previous
import functools

import jax
import jax.numpy as jnp
from jax import lax
from jax.experimental import pallas as pl
from jax.experimental.pallas import tpu as pltpu
from jax.experimental.pallas import tpu_sc as plsc

W = 2048
H = 2048
C = 4
N = 2000000

NC = 2
NS = 16
NW = NC * NS
VEC = 16

CHUNK = 2000
NVEC = CHUNK // VEC
NCHUNKS = N // CHUNK
ITERS = (NCHUNKS + NW - 1) // NW


def _search(c_ref, q, n):
  lo = jnp.zeros((VEC,), jnp.int32)
  step = n >> 1
  while step:
    m = lo + step
    v = plsc.load_gather(c_ref, [m - 1])
    lo = jnp.where(v <= q, m, lo)
    step >>= 1
  cmax = plsc.load_gather(c_ref, [jnp.full((VEC,), n - 1, jnp.int32)])
  cnt = jnp.where(cmax <= q, n, lo)
  xl = cnt - 1
  valid = (xl >= 0) & (xl <= n - 2)
  xlc = jnp.clip(xl, 0, n - 2)
  cl = plsc.load_gather(c_ref, [xlc])
  cu = plsc.load_gather(c_ref, [xlc + 1])
  rd = 1.0 / (cu - cl)
  return xlc, (cu - q) * rd, (q - cl) * rd, valid


def _make_kernel():
  mesh = plsc.VectorSubcoreMesh(core_axis_name="c", subcore_axis_name="s")

  @functools.partial(
      pl.kernel,
      out_type=[jax.ShapeDtypeStruct((N,), jnp.float32) for _ in range(C)],
      mesh=mesh,
      compiler_params=pltpu.CompilerParams(
          needs_layout_passes=False, use_tc_tiling_on_sc=False),
      scratch_types=[
          pltpu.VMEM((W,), jnp.float32),
          pltpu.VMEM((H,), jnp.float32),
          pltpu.VMEM((CHUNK,), jnp.float32),
          pltpu.VMEM((CHUNK,), jnp.float32),
          pltpu.VMEM((CHUNK,), jnp.int32),
          pltpu.VMEM((CHUNK,), jnp.int32),
          pltpu.VMEM((CHUNK,), jnp.float32),
          pltpu.VMEM((CHUNK,), jnp.float32),
          pltpu.VMEM((CHUNK,), jnp.float32),
          pltpu.VMEM((CHUNK,), jnp.float32),
          pltpu.VMEM((CHUNK,), jnp.float32),
          pltpu.VMEM((CHUNK, 8), jnp.float32),
          pltpu.VMEM((CHUNK, 8), jnp.float32),
          pltpu.VMEM((C, CHUNK), jnp.float32),
          pltpu.SemaphoreType.DMA,
      ],
  )
  def kern(cx_hbm, cy_hbm, xq_hbm, yq_hbm, rows8_hbm,
           o0_hbm, o1_hbm, o2_hbm, o3_hbm,
           cx, cy, xq, yq, idx0, idx1, wxl, wxu, wyl, wyu, msk, g0, g1, ob,
           sem):
    out_hbms = (o0_hbm, o1_hbm, o2_hbm, o3_hbm)
    wid = lax.axis_index("s") * NC + lax.axis_index("c")
    pltpu.sync_copy(cx_hbm, cx)
    pltpu.sync_copy(cy_hbm, cy)

    def chunk_body(i, _):
      cid = i * NW + wid

      @pl.when(cid < NCHUNKS)
      def _():
        base = cid * CHUNK
        pltpu.sync_copy(xq_hbm.at[pl.ds(base, CHUNK)], xq)
        pltpu.sync_copy(yq_hbm.at[pl.ds(base, CHUNK)], yq)

        def vec_body(v, _):
          off = v * VEC
          qx = xq[pl.ds(off, VEC)]
          qy = yq[pl.ds(off, VEC)]
          xlc, xw_l, xw_u, mx = _search(cx, qx, W)
          ylc, yw_l, yw_u, my = _search(cy, qy, H)
          p0 = ylc * W + xlc
          idx0[pl.ds(off, VEC)] = p0
          idx1[pl.ds(off, VEC)] = p0 + W
          wxl[pl.ds(off, VEC)] = xw_l
          wxu[pl.ds(off, VEC)] = xw_u
          wyl[pl.ds(off, VEC)] = yw_l
          wyu[pl.ds(off, VEC)] = yw_u
          msk[pl.ds(off, VEC)] = jnp.where(mx & my, 1.0, 0.0)
          return 0

        lax.fori_loop(0, NVEC, vec_body, 0)

        cp0 = pltpu.async_copy(rows8_hbm.at[idx0], g0, sem)
        cp1 = pltpu.async_copy(rows8_hbm.at[idx1], g1, sem)
        cp0.wait()
        cp1.wait()

        lanes = lax.iota(jnp.int32, VEC)

        def red_body(v, _):
          off = v * VEC
          qidx = lanes + off
          axl = wxl[pl.ds(off, VEC)]
          axu = wxu[pl.ds(off, VEC)]
          ayl = wyl[pl.ds(off, VEC)]
          ayu = wyu[pl.ds(off, VEC)]
          m = msk[pl.ds(off, VEC)]
          for c in range(C):
            cf = jnp.full((VEC,), c, jnp.int32)
            cf4 = jnp.full((VEC,), c + 4, jnp.int32)
            r00 = plsc.load_gather(g0, [qidx, cf])
            r01 = plsc.load_gather(g0, [qidx, cf4])
            r10 = plsc.load_gather(g1, [qidx, cf])
            r11 = plsc.load_gather(g1, [qidx, cf4])
            o = ayl * (axl * r00 + axu * r01) + ayu * (axl * r10 + axu * r11)
            o = jnp.where(m != 0.0, o, 0.0)
            ob[c, pl.ds(off, VEC)] = o
          return 0

        lax.fori_loop(0, NVEC, red_body, 0)

        for c in range(C):
          pltpu.sync_copy(ob.at[c], out_hbms[c].at[pl.ds(base, CHUNK)])

      return 0

    lax.fori_loop(0, ITERS, chunk_body, 0)

  return kern


_interp = _make_kernel()

RBLK = 8192
RGRID = -(-N // RBLK)


def _retile_body(i0, i1, i2, i3, o):
  rows = [x[...].reshape(1, RBLK) for x in (i0, i1, i2, i3)]
  o[...] = jnp.concatenate(rows, axis=0)


def _retile(chans):
  return pl.pallas_call(
      _retile_body,
      out_shape=jax.ShapeDtypeStruct((C, N), jnp.float32),
      grid=(RGRID,),
      in_specs=[pl.BlockSpec((RBLK,), lambda j: (j,)) for _ in range(C)],
      out_specs=pl.BlockSpec((C, RBLK), lambda j: (0, j)),
  )(*chans)


@jax.jit
def kernel(x_coords, y_coords, x_query, y_query, z):
  zrs = z.reshape(C, H * W).T
  rows8 = jnp.concatenate([zrs, jnp.roll(zrs, -1, axis=0)], axis=1)
  chans = _interp(x_coords, y_coords, x_query, y_query, rows8)
  return _retile(chans)


# speedup vs baseline: 147.9695x; 1.1452x over previous
import functools

import jax
import jax.numpy as jnp
from jax import lax
from jax.experimental import pallas as pl
from jax.experimental.pallas import tpu as pltpu
from jax.experimental.pallas import tpu_sc as plsc

W = 2048
H = 2048
C = 4
N = 2000000

NC = 2
NS = 16
NW = NC * NS
VEC = 16

CHUNK = 2000
NVEC = CHUNK // VEC
NCHUNKS = N // CHUNK
ITERS = (NCHUNKS + NW - 1) // NW


def _search(c_ref, q, n):
  lo = jnp.zeros((VEC,), jnp.int32)
  step = n >> 1
  while step:
    m = lo + step
    v = plsc.load_gather(c_ref, [m - 1])
    lo = jnp.where(v <= q, m, lo)
    step >>= 1
  cmax = plsc.load_gather(c_ref, [jnp.full((VEC,), n - 1, jnp.int32)])
  cnt = jnp.where(cmax <= q, n, lo)
  xl = cnt - 1
  valid = (xl >= 0) & (xl <= n - 2)
  xlc = jnp.clip(xl, 0, n - 2)
  cl = plsc.load_gather(c_ref, [xlc])
  cu = plsc.load_gather(c_ref, [xlc + 1])
  rd = 1.0 / (cu - cl)
  return xlc, (cu - q) * rd, (q - cl) * rd, valid


def _make_kernel():
  mesh = plsc.VectorSubcoreMesh(core_axis_name="c", subcore_axis_name="s")

  @functools.partial(
      pl.kernel,
      out_type=[jax.ShapeDtypeStruct((N,), jnp.float32) for _ in range(C)],
      mesh=mesh,
      compiler_params=pltpu.CompilerParams(
          needs_layout_passes=False, use_tc_tiling_on_sc=False),
      scratch_types=[
          pltpu.VMEM((W,), jnp.float32),
          pltpu.VMEM((H,), jnp.float32),
          pltpu.VMEM((CHUNK,), jnp.float32),
          pltpu.VMEM((CHUNK,), jnp.float32),
          pltpu.VMEM((CHUNK,), jnp.int32),
          pltpu.VMEM((CHUNK,), jnp.int32),
          pltpu.VMEM((CHUNK,), jnp.float32),
          pltpu.VMEM((CHUNK,), jnp.float32),
          pltpu.VMEM((CHUNK,), jnp.float32),
          pltpu.VMEM((CHUNK,), jnp.float32),
          pltpu.VMEM((CHUNK,), jnp.float32),
          pltpu.VMEM((CHUNK, 8), jnp.float32),
          pltpu.VMEM((CHUNK, 8), jnp.float32),
          pltpu.VMEM((C, CHUNK), jnp.float32),
          pltpu.SemaphoreType.DMA,
      ],
  )
  def kern(cx_hbm, cy_hbm, xq_hbm, yq_hbm, rows8_hbm,
           o0_hbm, o1_hbm, o2_hbm, o3_hbm,
           cx, cy, xq, yq, idx0, idx1, wxl, wxu, wyl, wyu, msk, g0, g1, ob,
           sem):
    out_hbms = (o0_hbm, o1_hbm, o2_hbm, o3_hbm)
    wid = lax.axis_index("s") * NC + lax.axis_index("c")
    pltpu.sync_copy(cx_hbm, cx)
    pltpu.sync_copy(cy_hbm, cy)

    def chunk_body(i, _):
      cid = i * NW + wid

      @pl.when(cid < NCHUNKS)
      def _():
        base = cid * CHUNK
        pltpu.sync_copy(xq_hbm.at[pl.ds(base, CHUNK)], xq)
        pltpu.sync_copy(yq_hbm.at[pl.ds(base, CHUNK)], yq)

        @plsc.parallel_loop(0, NVEC, unroll=4)
        def _(v):
          off = v * VEC
          qx = xq[pl.ds(off, VEC)]
          qy = yq[pl.ds(off, VEC)]
          xlc, xw_l, xw_u, mx = _search(cx, qx, W)
          ylc, yw_l, yw_u, my = _search(cy, qy, H)
          p0 = ylc * W + xlc
          idx0[pl.ds(off, VEC)] = p0
          idx1[pl.ds(off, VEC)] = p0 + W
          wxl[pl.ds(off, VEC)] = xw_l
          wxu[pl.ds(off, VEC)] = xw_u
          wyl[pl.ds(off, VEC)] = yw_l
          wyu[pl.ds(off, VEC)] = yw_u
          msk[pl.ds(off, VEC)] = jnp.where(mx & my, 1.0, 0.0)

        cp0 = pltpu.async_copy(rows8_hbm.at[idx0], g0, sem)
        cp1 = pltpu.async_copy(rows8_hbm.at[idx1], g1, sem)
        cp0.wait()
        cp1.wait()

        lanes = lax.iota(jnp.int32, VEC)

        @plsc.parallel_loop(0, NVEC, unroll=4)
        def _(v):
          off = v * VEC
          qidx = lanes + off
          axl = wxl[pl.ds(off, VEC)]
          axu = wxu[pl.ds(off, VEC)]
          ayl = wyl[pl.ds(off, VEC)]
          ayu = wyu[pl.ds(off, VEC)]
          m = msk[pl.ds(off, VEC)]
          for c in range(C):
            cf = jnp.full((VEC,), c, jnp.int32)
            cf4 = jnp.full((VEC,), c + 4, jnp.int32)
            r00 = plsc.load_gather(g0, [qidx, cf])
            r01 = plsc.load_gather(g0, [qidx, cf4])
            r10 = plsc.load_gather(g1, [qidx, cf])
            r11 = plsc.load_gather(g1, [qidx, cf4])
            o = ayl * (axl * r00 + axu * r01) + ayu * (axl * r10 + axu * r11)
            o = jnp.where(m != 0.0, o, 0.0)
            ob[c, pl.ds(off, VEC)] = o

        for c in range(C):
          pltpu.sync_copy(ob.at[c], out_hbms[c].at[pl.ds(base, CHUNK)])

      return 0

    lax.fori_loop(0, ITERS, chunk_body, 0)

  return kern


_interp = _make_kernel()

RBLK = 8192
RGRID = -(-N // RBLK)


def _retile_body(i0, i1, i2, i3, o):
  rows = [x[...].reshape(1, RBLK) for x in (i0, i1, i2, i3)]
  o[...] = jnp.concatenate(rows, axis=0)


def _retile(chans):
  return pl.pallas_call(
      _retile_body,
      out_shape=jax.ShapeDtypeStruct((C, N), jnp.float32),
      grid=(RGRID,),
      in_specs=[pl.BlockSpec((RBLK,), lambda j: (j,)) for _ in range(C)],
      out_specs=pl.BlockSpec((C, RBLK), lambda j: (0, j)),
  )(*chans)


@jax.jit
def kernel(x_coords, y_coords, x_query, y_query, z):
  zrs = z.reshape(C, H * W).T
  rows8 = jnp.concatenate([zrs, jnp.roll(zrs, -1, axis=0)], axis=1)
  chans = _interp(x_coords, y_coords, x_query, y_query, rows8)
  return _retile(chans)


# speedup vs baseline: 307.6783x; 2.0793x over previous
import functools

import jax
import jax.numpy as jnp
from jax import lax
from jax.experimental import pallas as pl
from jax.experimental.pallas import tpu as pltpu
from jax.experimental.pallas import tpu_sc as plsc

W = 2048
H = 2048
C = 4
N = 2000000

NC = 2
NS = 16
NW = NC * NS
VEC = 16

CHUNK = 2000
NVEC = CHUNK // VEC
NCHUNKS = N // CHUNK
ITERS = (NCHUNKS + NW - 1) // NW


def _search(c_ref, q, n):
  lo = jnp.zeros((VEC,), jnp.int32)
  step = n >> 1
  while step:
    m = lo + step
    v = plsc.load_gather(c_ref, [m - 1])
    lo = jnp.where(v <= q, m, lo)
    step >>= 1
  cmax = plsc.load_gather(c_ref, [jnp.full((VEC,), n - 1, jnp.int32)])
  cnt = jnp.where(cmax <= q, n, lo)
  xl = cnt - 1
  valid = (xl >= 0) & (xl <= n - 2)
  xlc = jnp.clip(xl, 0, n - 2)
  cl = plsc.load_gather(c_ref, [xlc])
  cu = plsc.load_gather(c_ref, [xlc + 1])
  rd = 1.0 / (cu - cl)
  return xlc, (cu - q) * rd, (q - cl) * rd, valid


SEGP = 2048
PSEG = (H * W) // NW // SEGP


def _make_prep():
  mesh = plsc.VectorSubcoreMesh(core_axis_name="c", subcore_axis_name="s")

  @functools.partial(
      pl.kernel,
      out_type=jax.ShapeDtypeStruct((H * W, 8), jnp.float32),
      mesh=mesh,
      compiler_params=pltpu.CompilerParams(
          needs_layout_passes=False, use_tc_tiling_on_sc=False),
      scratch_types=[
          [pltpu.VMEM((SEGP + 8,), jnp.float32) for _ in range(C)],
          pltpu.VMEM((SEGP, 8), jnp.float32),
      ],
  )
  def prep(zl_hbm, rows_hbm, zbufs, obuf):
    wid = lax.axis_index("s") * NC + lax.axis_index("c")
    p_lo = wid * (PSEG * SEGP)
    lanes = lax.iota(jnp.int32, VEC)

    def seg_body(s, _):
      pbase = p_lo + s * SEGP
      for c in range(C):
        src = c * (H * W) + pbase
        if c == C - 1:
          is_edge = pbase == (H * W - SEGP)

          @pl.when(is_edge)
          def _():
            pltpu.sync_copy(zl_hbm.at[pl.ds(src, SEGP)],
                            zbufs[c].at[pl.ds(0, SEGP)])

          @pl.when(jnp.logical_not(is_edge))
          def _():
            pltpu.sync_copy(zl_hbm.at[pl.ds(src, SEGP + 8)], zbufs[c])
        else:
          pltpu.sync_copy(zl_hbm.at[pl.ds(src, SEGP + 8)], zbufs[c])

      @plsc.parallel_loop(0, SEGP // VEC, unroll=4)
      def _(i):
        row = i * VEC + lanes
        for c in range(C):
          v0 = zbufs[c][pl.ds(i * VEC, VEC)]
          v1 = zbufs[c][pl.ds(i * VEC + 1, VEC)]
          plsc.store_scatter(obuf, [row, jnp.full((VEC,), c, jnp.int32)], v0)
          plsc.store_scatter(obuf, [row, jnp.full((VEC,), c + 4, jnp.int32)],
                             v1)

      pltpu.sync_copy(obuf, rows_hbm.at[pl.ds(pbase, SEGP)])
      return 0

    lax.fori_loop(0, PSEG, seg_body, 0)

  return prep


_prep = _make_prep()


def _make_kernel():
  mesh = plsc.VectorSubcoreMesh(core_axis_name="c", subcore_axis_name="s")

  @functools.partial(
      pl.kernel,
      out_type=[jax.ShapeDtypeStruct((N,), jnp.float32) for _ in range(C)],
      mesh=mesh,
      compiler_params=pltpu.CompilerParams(
          needs_layout_passes=False, use_tc_tiling_on_sc=False),
      scratch_types=[
          pltpu.VMEM((W,), jnp.float32),
          pltpu.VMEM((H,), jnp.float32),
          pltpu.VMEM((CHUNK,), jnp.float32),
          pltpu.VMEM((CHUNK,), jnp.float32),
          pltpu.VMEM((CHUNK,), jnp.int32),
          pltpu.VMEM((CHUNK,), jnp.int32),
          pltpu.VMEM((CHUNK,), jnp.float32),
          pltpu.VMEM((CHUNK,), jnp.float32),
          pltpu.VMEM((CHUNK,), jnp.float32),
          pltpu.VMEM((CHUNK,), jnp.float32),
          pltpu.VMEM((CHUNK,), jnp.float32),
          pltpu.VMEM((CHUNK, 8), jnp.float32),
          pltpu.VMEM((CHUNK, 8), jnp.float32),
          pltpu.VMEM((C, CHUNK), jnp.float32),
          pltpu.SemaphoreType.DMA,
      ],
  )
  def kern(cx_hbm, cy_hbm, xq_hbm, yq_hbm, rows8_hbm,
           o0_hbm, o1_hbm, o2_hbm, o3_hbm,
           cx, cy, xq, yq, idx0, idx1, wxl, wxu, wyl, wyu, msk, g0, g1, ob,
           sem):
    out_hbms = (o0_hbm, o1_hbm, o2_hbm, o3_hbm)
    wid = lax.axis_index("s") * NC + lax.axis_index("c")
    pltpu.sync_copy(cx_hbm, cx)
    pltpu.sync_copy(cy_hbm, cy)

    def chunk_body(i, _):
      cid = i * NW + wid

      @pl.when(cid < NCHUNKS)
      def _():
        base = cid * CHUNK
        pltpu.sync_copy(xq_hbm.at[pl.ds(base, CHUNK)], xq)
        pltpu.sync_copy(yq_hbm.at[pl.ds(base, CHUNK)], yq)

        @plsc.parallel_loop(0, NVEC, unroll=4)
        def _(v):
          off = v * VEC
          qx = xq[pl.ds(off, VEC)]
          qy = yq[pl.ds(off, VEC)]
          xlc, xw_l, xw_u, mx = _search(cx, qx, W)
          ylc, yw_l, yw_u, my = _search(cy, qy, H)
          p0 = ylc * W + xlc
          idx0[pl.ds(off, VEC)] = p0
          idx1[pl.ds(off, VEC)] = p0 + W
          wxl[pl.ds(off, VEC)] = xw_l
          wxu[pl.ds(off, VEC)] = xw_u
          wyl[pl.ds(off, VEC)] = yw_l
          wyu[pl.ds(off, VEC)] = yw_u
          msk[pl.ds(off, VEC)] = jnp.where(mx & my, 1.0, 0.0)

        cp0 = pltpu.async_copy(rows8_hbm.at[idx0], g0, sem)
        cp1 = pltpu.async_copy(rows8_hbm.at[idx1], g1, sem)
        cp0.wait()
        cp1.wait()

        lanes = lax.iota(jnp.int32, VEC)

        @plsc.parallel_loop(0, NVEC, unroll=4)
        def _(v):
          off = v * VEC
          qidx = lanes + off
          axl = wxl[pl.ds(off, VEC)]
          axu = wxu[pl.ds(off, VEC)]
          ayl = wyl[pl.ds(off, VEC)]
          ayu = wyu[pl.ds(off, VEC)]
          m = msk[pl.ds(off, VEC)]
          for c in range(C):
            cf = jnp.full((VEC,), c, jnp.int32)
            cf4 = jnp.full((VEC,), c + 4, jnp.int32)
            r00 = plsc.load_gather(g0, [qidx, cf])
            r01 = plsc.load_gather(g0, [qidx, cf4])
            r10 = plsc.load_gather(g1, [qidx, cf])
            r11 = plsc.load_gather(g1, [qidx, cf4])
            o = ayl * (axl * r00 + axu * r01) + ayu * (axl * r10 + axu * r11)
            o = jnp.where(m != 0.0, o, 0.0)
            ob[c, pl.ds(off, VEC)] = o

        for c in range(C):
          pltpu.sync_copy(ob.at[c], out_hbms[c].at[pl.ds(base, CHUNK)])

      return 0

    lax.fori_loop(0, ITERS, chunk_body, 0)

  return kern


_interp = _make_kernel()

RBLK = 8192
RGRID = -(-N // RBLK)


def _retile_body(i0, i1, i2, i3, o):
  rows = [x[...].reshape(1, RBLK) for x in (i0, i1, i2, i3)]
  o[...] = jnp.concatenate(rows, axis=0)


def _retile(chans):
  return pl.pallas_call(
      _retile_body,
      out_shape=jax.ShapeDtypeStruct((C, N), jnp.float32),
      grid=(RGRID,),
      in_specs=[pl.BlockSpec((RBLK,), lambda j: (j,)) for _ in range(C)],
      out_specs=pl.BlockSpec((C, RBLK), lambda j: (0, j)),
  )(*chans)


@jax.jit
def kernel(x_coords, y_coords, x_query, y_query, z):
  rows8 = _prep(z.reshape(C * H * W))
  chans = _interp(x_coords, y_coords, x_query, y_query, rows8)
  return _retile(chans)
